# baseline (speedup 1.0000x reference)
import functools

import jax
import jax.numpy as jnp
from jax import lax
from jax.experimental import pallas as pl
from jax.experimental.pallas import tpu as pltpu
from jax.experimental.pallas import tpu_sc as plsc

DIM = 32
BATCH = 16384
NC = 2
NS = 16
L = 16
NW = NC * NS
BPW = BATCH // NW
GROUPS = BPW // L


def _bpr_body(uid_hbm, iid_hbm, uemb_hbm, iemb_hbm, ub_hbm, ib_hbm, out_hbm,
              uid_v, iid_v, urows, irows, ubv, ibv, dotv,
              sem_u, sem_i, sem_ub, sem_ib):
    wid = lax.axis_index("s") * NC + lax.axis_index("c")
    base = wid * BPW

    pltpu.sync_copy(uid_hbm.at[pl.ds(base, BPW)], uid_v)
    pltpu.sync_copy(iid_hbm.at[pl.ds(base, BPW)], iid_v)

    cu = pltpu.async_copy(uemb_hbm.at[uid_v], urows, sem_u)
    ci = pltpu.async_copy(iemb_hbm.at[iid_v], irows, sem_i)
    cub = pltpu.async_copy(ub_hbm.at[uid_v], ubv, sem_ub)
    cib = pltpu.async_copy(ib_hbm.at[iid_v], ibv, sem_ib)
    cu.wait()
    ci.wait()
    cub.wait()
    cib.wait()

    iota = lax.iota(jnp.int32, L)

    def group(g, carry):
        row = g * L + iota
        acc = ubv[pl.ds(g * L, L)] + ibv[pl.ds(g * L, L)]
        for d in range(DIM):
            col = jnp.full((L,), d, jnp.int32)
            acc = acc + (plsc.load_gather(urows, [row, col])
                         * plsc.load_gather(irows, [row, col]))
        dotv[pl.ds(g * L, L)] = acc
        return carry

    lax.fori_loop(0, GROUPS, group, 0)
    pltpu.sync_copy(dotv, out_hbm.at[pl.ds(base, BPW)])


@jax.jit
def kernel(user_ids, item_ids, user_emb, item_emb, user_bias, item_bias):
    uid = user_ids.astype(jnp.int32)
    iid = item_ids.astype(jnp.int32)
    mesh = plsc.VectorSubcoreMesh(core_axis_name="c", subcore_axis_name="s")
    run = functools.partial(
        pl.kernel,
        mesh=mesh,
        compiler_params=pltpu.CompilerParams(
            needs_layout_passes=False, use_tc_tiling_on_sc=False),
        out_type=jax.ShapeDtypeStruct((BATCH,), jnp.float32),
        scratch_types=[
            pltpu.VMEM((BPW,), jnp.int32),
            pltpu.VMEM((BPW,), jnp.int32),
            pltpu.VMEM((BPW, DIM), jnp.float32),
            pltpu.VMEM((BPW, DIM), jnp.float32),
            pltpu.VMEM((BPW,), jnp.float32),
            pltpu.VMEM((BPW,), jnp.float32),
            pltpu.VMEM((BPW,), jnp.float32),
            pltpu.SemaphoreType.DMA,
            pltpu.SemaphoreType.DMA,
            pltpu.SemaphoreType.DMA,
            pltpu.SemaphoreType.DMA,
        ],
    )(_bpr_body)
    return run(uid, iid, user_emb, item_emb,
               user_bias.reshape(-1), item_bias.reshape(-1))
